# speedup vs baseline: 1.4173x; 1.4173x over previous
import functools

import jax
import jax.numpy as jnp
from jax.experimental import pallas as pl
from jax.experimental.pallas import tpu as pltpu


def _body(x_ref, wg_ref, bg_ref, w1_ref, b1_ref, w2_ref, b2_ref, out_ref,
          g_scr, vacc, *, n_experts):
    j = pl.program_id(1)
    xb = x_ref[...]
    xb16 = xb.astype(jnp.bfloat16)

    @pl.when(j == 0)
    def _():
        z = jnp.dot(xb16, wg_ref[...].astype(jnp.bfloat16),
                    preferred_element_type=jnp.float32) + bg_ref[...]
        g_scr[...] = jnp.maximum(z, 0.0) + jnp.log1p(jnp.exp(-jnp.abs(z)))
        vacc[...] = jnp.zeros_like(vacc)

    h = jnp.dot(xb16, w1_ref[0].astype(jnp.bfloat16),
                preferred_element_type=jnp.float32)
    h = jnp.maximum(h + b1_ref[0], 0.0)
    o = jnp.sum(h * w2_ref[0], axis=1, keepdims=True) + b2_ref[0]

    g = g_scr[...]
    mask = (jax.lax.broadcasted_iota(jnp.int32, (1, n_experts), 1) == j)
    gj = jnp.sum(g * mask.astype(jnp.float32), axis=1, keepdims=True)
    vacc[...] += gj * o

    @pl.when(j == n_experts - 1)
    def _():
        summ = jnp.sum(g, axis=1, keepdims=True)
        out_ref[...] = vacc[...] / summ


@jax.jit
def kernel(x, Wg, bg, W1, b1, W2, b2):
    N, D = x.shape
    E, _, F = W1.shape
    TN = 1024
    grid = (N // TN, E)

    bgr = bg.reshape(1, E)
    b1r = b1.reshape(E, 1, F)
    w2r = W2.reshape(E, 1, F)
    b2r = b2.reshape(E, 1, 1)

    out = pl.pallas_call(
        functools.partial(_body, n_experts=E),
        grid=grid,
        in_specs=[
            pl.BlockSpec((TN, D), lambda i, j: (i, 0)),
            pl.BlockSpec((D, E), lambda i, j: (0, 0)),
            pl.BlockSpec((1, E), lambda i, j: (0, 0)),
            pl.BlockSpec((1, D, F), lambda i, j: (j, 0, 0)),
            pl.BlockSpec((1, 1, F), lambda i, j: (j, 0, 0)),
            pl.BlockSpec((1, 1, F), lambda i, j: (j, 0, 0)),
            pl.BlockSpec((1, 1, 1), lambda i, j: (j, 0, 0)),
        ],
        out_specs=pl.BlockSpec((TN, 1), lambda i, j: (i, 0)),
        out_shape=jax.ShapeDtypeStruct((N, 1), jnp.float32),
        scratch_shapes=[
            pltpu.VMEM((TN, E), jnp.float32),
            pltpu.VMEM((TN, 1), jnp.float32),
        ],
        compiler_params=pltpu.CompilerParams(
            dimension_semantics=("parallel", "arbitrary")),
    )(x, Wg, bgr, W1, b1r, w2r, b2r)
    return out.reshape(-1)
